# baseline (speedup 1.0000x reference)
import functools

import jax
import jax.numpy as jnp
from jax import lax
from jax.experimental import pallas as pl
from jax.experimental.pallas import tpu as pltpu
from jax.experimental.pallas import tpu_sc as plsc

_COLS = 2048
_SBLK = 1024
_CHUNK = 512
_NBUF = 2


def _detile_body(x_ref, o_ref):
  y = x_ref[...].T
  half = y.shape[0] // 2
  o_ref[...] = jnp.concatenate([y[0:half], y[half:]], axis=1)


@functools.lru_cache(maxsize=None)
def _build_detile(V, D, cols):
  grid = -(-V // cols)
  return pl.pallas_call(
      _detile_body,
      grid=(grid,),
      in_specs=[pl.BlockSpec((D, cols), lambda i: (0, i))],
      out_specs=pl.BlockSpec((cols // 2, 128), lambda i: (i, 0)),
      out_shape=jax.ShapeDtypeStruct((grid * cols // 2, 128), jnp.float32),
  )


def _idx_body(x_ref, o_ref):
  x = x_ref[...]
  o_ref[...] = jnp.concatenate(
      [x[:, 128 * t:128 * (t + 1)][:, None, :] for t in range(16)], axis=1)


@functools.lru_cache(maxsize=None)
def _build_idx(S, W):
  grid_w = -(-W // 8)
  grid_s = S // 2048
  return pl.pallas_call(
      _idx_body,
      grid=(grid_w, grid_s),
      in_specs=[pl.BlockSpec((8, 2048), lambda w, j: (w, j))],
      out_specs=pl.BlockSpec((8, 16, 128), lambda w, j: (w, j, 0)),
      out_shape=jax.ShapeDtypeStruct((grid_w * 8, S // 128, 128), jnp.int32),
  )


@functools.lru_cache(maxsize=None)
def _build_gather(B, V, D, chunk):
  NC, NS = 2, 16
  NW = NC * NS
  b_per_w = B // NW
  n_chunk = b_per_w // chunk
  raw_rows = b_per_w // 128
  ready_rows = b_per_w // chunk
  assert n_chunk % _NBUF == 0

  mesh = plsc.VectorSubcoreMesh(core_axis_name="c", subcore_axis_name="s")

  @functools.partial(
      pl.kernel,
      mesh=mesh,
      compiler_params=pltpu.CompilerParams(
          use_tc_tiling_on_sc=False, needs_layout_passes=False),
      out_type=jax.ShapeDtypeStruct((B, D), jnp.float32),
      scratch_types=[
          pltpu.VMEM((raw_rows, 128), jnp.int32),
          pltpu.VMEM((ready_rows, chunk), jnp.int32),
          pltpu.VMEM((_NBUF * chunk, D), jnp.float32),
          pltpu.SemaphoreType.DMA((_NBUF,)),
          pltpu.SemaphoreType.DMA((_NBUF,)),
      ],
  )
  def gather_kernel(idx_hbm, table_hbm, out_hbm, raw_v, idx_v, rows_v, gsem,
                    osem):
    wid = lax.axis_index("s") * NC + lax.axis_index("c")
    out_base_w = wid * b_per_w

    pltpu.sync_copy(idx_hbm.at[pl.ds(wid * raw_rows, raw_rows)], raw_v)

    i16 = lax.iota(jnp.int32, 16)
    pattern = ((i16 & 1) << 9) + (i16 >> 1)
    vregs_per_row = chunk // 16

    def permute_row(r, carry):
      for v in range(vregs_per_row):
        k0 = r * chunk + v * 16
        w0 = k0 & ~(_SBLK - 1)
        src = pattern + (w0 + ((k0 - w0) >> 1))
        val = plsc.load_gather(raw_v, [src >> 7, src & 127])
        idx_v[r, pl.ds(v * 16, 16)] = val
      return carry

    lax.fori_loop(0, ready_rows, permute_row, 0)

    def fire_gather(g, b):
      pltpu.async_copy(table_hbm.at[idx_v.at[g]],
                       rows_v.at[pl.ds(b * chunk, chunk)], gsem.at[b])

    def drain_gather(b):
      pltpu.make_async_copy(table_hbm.at[pl.ds(0, chunk)],
                            rows_v.at[pl.ds(b * chunk, chunk)],
                            gsem.at[b]).wait()

    def wait_outwrite(b):
      pltpu.make_async_copy(rows_v.at[pl.ds(b * chunk, chunk)],
                            out_hbm.at[pl.ds(out_base_w, chunk)],
                            osem.at[b]).wait()

    for b in range(_NBUF):
      fire_gather(b, b)

    def body(t, carry):
      for b in range(_NBUF):
        g = _NBUF * t + b
        drain_gather(b)
        pltpu.async_copy(rows_v.at[pl.ds(b * chunk, chunk)],
                         out_hbm.at[pl.ds(out_base_w + g * chunk, chunk)],
                         osem.at[b])
      for b in range(_NBUF):
        g_next = _NBUF * t + b + _NBUF

        @pl.when(g_next < n_chunk)
        def _():
          wait_outwrite(b)
          fire_gather(g_next, b)

      return carry

    lax.fori_loop(0, n_chunk // _NBUF, body, 0)
    for b in range(_NBUF):
      wait_outwrite(b)

  return gather_kernel


def _retile_body(x_ref, o_ref):
  t = x_ref[...][0].T
  half = t.shape[1]
  o_ref[0, :, 0:half] = t[0:64]
  o_ref[0, :, half:2 * half] = t[64:128]


@functools.lru_cache(maxsize=None)
def _build_retile(S, W, D, sblk):
  grid_s = S // sblk
  return pl.pallas_call(
      _retile_body,
      grid=(W, grid_s),
      in_specs=[
          pl.BlockSpec((1, sblk * D // 128, 128), lambda w, j: (w, j, 0))
      ],
      out_specs=pl.BlockSpec((1, D, sblk), lambda w, j: (w, 0, j)),
      out_shape=jax.ShapeDtypeStruct((W, D, S), jnp.float32),
  )


def kernel(input_sentence, word_embedding_weight):
  S, W = input_sentence.shape
  V, D = word_embedding_weight.shape
  B = S * W

  wt = word_embedding_weight.T
  t2d = _build_detile(V, D, _COLS)(wt)
  Vp = t2d.shape[0] * 2
  table_rm = t2d.reshape(Vp * D // 128 * 128).reshape(Vp, D)

  idxT = input_sentence.T.astype(jnp.int32)
  u = (jnp.bitwise_and(idxT, -_COLS)
       + ((idxT & (_COLS // 2 - 1)) << 1)
       + ((idxT >> 10) & 1))

  idx3 = _build_idx(S, W)(u)
  idx2d = idx3.reshape(idx3.shape[0] * S // 128 * 128).reshape(-1, 128)

  out_sc = _build_gather(B, Vp, D, _CHUNK)(idx2d, table_rm)

  out_v = out_sc.reshape(B * D).reshape(W, S * D // 128, 128)
  out_t = _build_retile(S, W, D, _SBLK)(out_v)
  return out_t.transpose(2, 0, 1)


# speedup vs baseline: 1.7449x; 1.7449x over previous
import functools

import jax
import jax.numpy as jnp
from jax import lax
from jax.experimental import pallas as pl
from jax.experimental.pallas import tpu as pltpu
from jax.experimental.pallas import tpu_sc as plsc

_COLS = 8192
_WIN = 1024
_SBLK = 8192
_CHUNK = 512
_NBUF = 2


def _detile_body(x_ref, o_ref):
  y = x_ref[...].T
  half = y.shape[0] // 2
  o_ref[...] = jnp.concatenate([y[0:half], y[half:]], axis=1)


@functools.lru_cache(maxsize=None)
def _build_detile(V, D, cols):
  grid = -(-V // cols)
  return pl.pallas_call(
      _detile_body,
      grid=(grid,),
      in_specs=[pl.BlockSpec((D, cols), lambda i: (0, i))],
      out_specs=pl.BlockSpec((cols // 2, 128), lambda i: (i, 0)),
      out_shape=jax.ShapeDtypeStruct((grid * cols // 2, 128), jnp.float32),
  )


def _idx_body(x_ref, o_ref):
  x = x_ref[...]
  o_ref[...] = jnp.concatenate(
      [x[:, 128 * t:128 * (t + 1)][:, None, :] for t in range(16)], axis=1)


@functools.lru_cache(maxsize=None)
def _build_idx(S, W):
  grid_w = -(-W // 8)
  grid_s = S // 2048
  return pl.pallas_call(
      _idx_body,
      grid=(grid_w, grid_s),
      in_specs=[pl.BlockSpec((8, 2048), lambda w, j: (w, j))],
      out_specs=pl.BlockSpec((8, 16, 128), lambda w, j: (w, j, 0)),
      out_shape=jax.ShapeDtypeStruct((grid_w * 8, S // 128, 128), jnp.int32),
  )


@functools.lru_cache(maxsize=None)
def _build_gather(B, V, D, chunk):
  NC, NS = 2, 16
  NW = NC * NS
  b_per_w = B // NW
  n_chunk = b_per_w // chunk
  raw_rows = b_per_w // 128
  ready_rows = b_per_w // chunk
  assert n_chunk % _NBUF == 0

  mesh = plsc.VectorSubcoreMesh(core_axis_name="c", subcore_axis_name="s")

  @functools.partial(
      pl.kernel,
      mesh=mesh,
      compiler_params=pltpu.CompilerParams(
          use_tc_tiling_on_sc=False, needs_layout_passes=False),
      out_type=jax.ShapeDtypeStruct((B, D), jnp.float32),
      scratch_types=[
          pltpu.VMEM((raw_rows, 128), jnp.int32),
          pltpu.VMEM((ready_rows, chunk), jnp.int32),
          pltpu.VMEM((_NBUF * chunk, D), jnp.float32),
          pltpu.SemaphoreType.DMA((_NBUF,)),
          pltpu.SemaphoreType.DMA((_NBUF,)),
      ],
  )
  def gather_kernel(idx_hbm, table_hbm, out_hbm, raw_v, idx_v, rows_v, gsem,
                    osem):
    wid = lax.axis_index("s") * NC + lax.axis_index("c")
    out_base_w = wid * b_per_w

    pltpu.sync_copy(idx_hbm.at[pl.ds(wid * raw_rows, raw_rows)], raw_v)

    i16 = lax.iota(jnp.int32, 16)
    pattern = ((i16 & 1) * (_WIN // 2)) + (i16 >> 1)
    vregs_per_row = chunk // 16

    def permute_row(r, carry):
      for v in range(vregs_per_row):
        k0 = r * chunk + v * 16
        w0 = k0 & ~(_WIN - 1)
        src = pattern + (w0 + ((k0 - w0) >> 1))
        val = plsc.load_gather(raw_v, [src >> 7, src & 127])
        idx_v[r, pl.ds(v * 16, 16)] = val
      return carry

    lax.fori_loop(0, ready_rows, permute_row, 0)

    def fire_gather(g, b):
      pltpu.async_copy(table_hbm.at[idx_v.at[g]],
                       rows_v.at[pl.ds(b * chunk, chunk)], gsem.at[b])

    def drain_gather(b):
      pltpu.make_async_copy(table_hbm.at[pl.ds(0, chunk)],
                            rows_v.at[pl.ds(b * chunk, chunk)],
                            gsem.at[b]).wait()

    def wait_outwrite(b):
      pltpu.make_async_copy(rows_v.at[pl.ds(b * chunk, chunk)],
                            out_hbm.at[pl.ds(out_base_w, chunk)],
                            osem.at[b]).wait()

    for b in range(_NBUF):
      fire_gather(b, b)

    def body(t, carry):
      for b in range(_NBUF):
        g = _NBUF * t + b
        drain_gather(b)
        pltpu.async_copy(rows_v.at[pl.ds(b * chunk, chunk)],
                         out_hbm.at[pl.ds(out_base_w + g * chunk, chunk)],
                         osem.at[b])
      for b in range(_NBUF):
        g_next = _NBUF * t + b + _NBUF

        @pl.when(g_next < n_chunk)
        def _():
          wait_outwrite(b)
          fire_gather(g_next, b)

      return carry

    lax.fori_loop(0, n_chunk // _NBUF, body, 0)
    for b in range(_NBUF):
      wait_outwrite(b)

  return gather_kernel


def _retile_body(x_ref, o_ref):
  x = x_ref[...][0]
  h = _WIN // 2
  for u in range(x.shape[0] // h):
    win = x[u * h:(u + 1) * h]
    o_ref[0, :, u * _WIN:u * _WIN + h] = win[:, 0:64].T
    o_ref[0, :, u * _WIN + h:(u + 1) * _WIN] = win[:, 64:128].T


@functools.lru_cache(maxsize=None)
def _build_retile(S, W, D, sblk):
  grid_s = S // sblk
  return pl.pallas_call(
      _retile_body,
      grid=(W, grid_s),
      in_specs=[
          pl.BlockSpec((1, sblk * D // 128, 128), lambda w, j: (w, j, 0))
      ],
      out_specs=pl.BlockSpec((1, D, sblk), lambda w, j: (w, 0, j)),
      out_shape=jax.ShapeDtypeStruct((W, D, S), jnp.float32),
  )


def kernel(input_sentence, word_embedding_weight):
  S, W = input_sentence.shape
  V, D = word_embedding_weight.shape
  B = S * W

  wt = word_embedding_weight.T
  t2d = _build_detile(V, D, _COLS)(wt)
  Vp = t2d.shape[0] * 2
  table_rm = t2d.reshape(Vp * D // 128 * 128).reshape(Vp, D)

  idxT = input_sentence.T.astype(jnp.int32)
  half_bits = (_COLS // 2).bit_length() - 1
  u = (jnp.bitwise_and(idxT, -_COLS)
       + ((idxT & (_COLS // 2 - 1)) << 1)
       + ((idxT >> half_bits) & 1))

  idx3 = _build_idx(S, W)(u)
  idx2d = idx3.reshape(idx3.shape[0] * S // 128 * 128).reshape(-1, 128)

  out_sc = _build_gather(B, Vp, D, _CHUNK)(idx2d, table_rm)

  out_v = out_sc.reshape(B * D).reshape(W, S * D // 128, 128)
  out_t = _build_retile(S, W, D, _SBLK)(out_v)
  return out_t.transpose(2, 0, 1)


# speedup vs baseline: 1.7494x; 1.0026x over previous
import functools

import jax
import jax.numpy as jnp
from jax import lax
from jax.experimental import pallas as pl
from jax.experimental.pallas import tpu as pltpu
from jax.experimental.pallas import tpu_sc as plsc

_COLS = 8192
_WIN = 1024
_SBLK = 8192
_CHUNK = 256
_NBUF = 4


def _detile_body(x_ref, o_ref):
  y = x_ref[...].T
  half = y.shape[0] // 2
  o_ref[...] = jnp.concatenate([y[0:half], y[half:]], axis=1)


@functools.lru_cache(maxsize=None)
def _build_detile(V, D, cols):
  grid = -(-V // cols)
  return pl.pallas_call(
      _detile_body,
      grid=(grid,),
      in_specs=[pl.BlockSpec((D, cols), lambda i: (0, i))],
      out_specs=pl.BlockSpec((cols // 2, 128), lambda i: (i, 0)),
      out_shape=jax.ShapeDtypeStruct((grid * cols // 2, 128), jnp.float32),
  )


def _idx_body(x_ref, o_ref):
  x = x_ref[...]
  o_ref[...] = jnp.concatenate(
      [x[:, 128 * t:128 * (t + 1)][:, None, :] for t in range(16)], axis=1)


@functools.lru_cache(maxsize=None)
def _build_idx(S, W):
  grid_w = -(-W // 8)
  grid_s = S // 2048
  return pl.pallas_call(
      _idx_body,
      grid=(grid_w, grid_s),
      in_specs=[pl.BlockSpec((8, 2048), lambda w, j: (w, j))],
      out_specs=pl.BlockSpec((8, 16, 128), lambda w, j: (w, j, 0)),
      out_shape=jax.ShapeDtypeStruct((grid_w * 8, S // 128, 128), jnp.int32),
  )


@functools.lru_cache(maxsize=None)
def _build_gather(B, V, D, chunk):
  NC, NS = 2, 16
  NW = NC * NS
  b_per_w = B // NW
  n_chunk = b_per_w // chunk
  raw_rows = b_per_w // 128
  ready_rows = b_per_w // chunk
  assert n_chunk % _NBUF == 0

  mesh = plsc.VectorSubcoreMesh(core_axis_name="c", subcore_axis_name="s")

  @functools.partial(
      pl.kernel,
      mesh=mesh,
      compiler_params=pltpu.CompilerParams(
          use_tc_tiling_on_sc=False, needs_layout_passes=False),
      out_type=jax.ShapeDtypeStruct((B, D), jnp.float32),
      scratch_types=[
          pltpu.VMEM((raw_rows, 128), jnp.int32),
          pltpu.VMEM((ready_rows, chunk), jnp.int32),
          pltpu.VMEM((_NBUF * chunk, D), jnp.float32),
          pltpu.SemaphoreType.DMA((_NBUF,)),
          pltpu.SemaphoreType.DMA((_NBUF,)),
      ],
  )
  def gather_kernel(idx_hbm, table_hbm, out_hbm, raw_v, idx_v, rows_v, gsem,
                    osem):
    wid = lax.axis_index("s") * NC + lax.axis_index("c")
    out_base_w = wid * b_per_w

    pltpu.sync_copy(idx_hbm.at[pl.ds(wid * raw_rows, raw_rows)], raw_v)

    i16 = lax.iota(jnp.int32, 16)
    pattern = ((i16 & 1) * (_WIN // 2)) + (i16 >> 1)
    vregs_per_row = chunk // 16

    def permute_row(r, carry):
      for v in range(vregs_per_row):
        k0 = r * chunk + v * 16
        w0 = k0 & ~(_WIN - 1)
        src = pattern + (w0 + ((k0 - w0) >> 1))
        val = plsc.load_gather(raw_v, [src >> 7, src & 127])
        idx_v[r, pl.ds(v * 16, 16)] = val
      return carry

    lax.fori_loop(0, ready_rows, permute_row, 0)

    def fire_gather(g, b):
      pltpu.async_copy(table_hbm.at[idx_v.at[g]],
                       rows_v.at[pl.ds(b * chunk, chunk)], gsem.at[b])

    def drain_gather(b):
      pltpu.make_async_copy(table_hbm.at[pl.ds(0, chunk)],
                            rows_v.at[pl.ds(b * chunk, chunk)],
                            gsem.at[b]).wait()

    def wait_outwrite(b):
      pltpu.make_async_copy(rows_v.at[pl.ds(b * chunk, chunk)],
                            out_hbm.at[pl.ds(out_base_w, chunk)],
                            osem.at[b]).wait()

    for b in range(_NBUF):
      fire_gather(b, b)

    def body(t, carry):
      for b in range(_NBUF):
        g = _NBUF * t + b
        drain_gather(b)
        pltpu.async_copy(rows_v.at[pl.ds(b * chunk, chunk)],
                         out_hbm.at[pl.ds(out_base_w + g * chunk, chunk)],
                         osem.at[b])
      for b in range(_NBUF):
        g_next = _NBUF * t + b + _NBUF

        @pl.when(g_next < n_chunk)
        def _():
          wait_outwrite(b)
          fire_gather(g_next, b)

      return carry

    lax.fori_loop(0, n_chunk // _NBUF, body, 0)
    for b in range(_NBUF):
      wait_outwrite(b)

  return gather_kernel


def _retile_body(x_ref, o_ref):
  x = x_ref[...][0]
  h = _WIN // 2
  for u in range(x.shape[0] // h):
    win = x[u * h:(u + 1) * h]
    o_ref[0, :, u * _WIN:u * _WIN + h] = win[:, 0:64].T
    o_ref[0, :, u * _WIN + h:(u + 1) * _WIN] = win[:, 64:128].T


@functools.lru_cache(maxsize=None)
def _build_retile(S, W, D, sblk):
  grid_s = S // sblk
  return pl.pallas_call(
      _retile_body,
      grid=(W, grid_s),
      in_specs=[
          pl.BlockSpec((1, sblk * D // 128, 128), lambda w, j: (w, j, 0))
      ],
      out_specs=pl.BlockSpec((1, D, sblk), lambda w, j: (w, 0, j)),
      out_shape=jax.ShapeDtypeStruct((W, D, S), jnp.float32),
  )


def kernel(input_sentence, word_embedding_weight):
  S, W = input_sentence.shape
  V, D = word_embedding_weight.shape
  B = S * W

  wt = word_embedding_weight.T
  t2d = _build_detile(V, D, _COLS)(wt)
  Vp = t2d.shape[0] * 2
  table_rm = t2d.reshape(Vp * D // 128 * 128).reshape(Vp, D)

  idxT = input_sentence.T.astype(jnp.int32)
  half_bits = (_COLS // 2).bit_length() - 1
  u = (jnp.bitwise_and(idxT, -_COLS)
       + ((idxT & (_COLS // 2 - 1)) << 1)
       + ((idxT >> half_bits) & 1))

  idx3 = _build_idx(S, W)(u)
  idx2d = idx3.reshape(idx3.shape[0] * S // 128 * 128).reshape(-1, 128)

  out_sc = _build_gather(B, Vp, D, _CHUNK)(idx2d, table_rm)

  out_v = out_sc.reshape(B * D).reshape(W, S * D // 128, 128)
  out_t = _build_retile(S, W, D, _SBLK)(out_v)
  return out_t.transpose(2, 0, 1)


# speedup vs baseline: 1.9184x; 1.0966x over previous
import functools

import jax
import jax.numpy as jnp
from jax import lax
from jax.experimental import pallas as pl
from jax.experimental.pallas import tpu as pltpu
from jax.experimental.pallas import tpu_sc as plsc

_COLS = 16384
_WIN = 1024
_SBLK = 16384
_CHUNK = 256
_NBUF = 4


def _detile_body(x_ref, o_ref):
  y = x_ref[...].T
  half = y.shape[0] // 2
  o_ref[...] = jnp.concatenate([y[0:half], y[half:]], axis=1)


@functools.lru_cache(maxsize=None)
def _build_detile(V, D, cols):
  grid = -(-V // cols)
  return pl.pallas_call(
      _detile_body,
      grid=(grid,),
      in_specs=[pl.BlockSpec((D, cols), lambda i: (0, i))],
      out_specs=pl.BlockSpec((cols // 2, 128), lambda i: (i, 0)),
      out_shape=jax.ShapeDtypeStruct((grid * cols // 2, 128), jnp.float32),
  )


def _idx_body(x_ref, o_ref):
  x = x_ref[...]
  o_ref[...] = jnp.concatenate(
      [x[:, 128 * t:128 * (t + 1)][:, None, :] for t in range(16)], axis=1)


@functools.lru_cache(maxsize=None)
def _build_idx(S, W):
  grid_w = -(-W // 8)
  grid_s = S // 2048
  return pl.pallas_call(
      _idx_body,
      grid=(grid_w, grid_s),
      in_specs=[pl.BlockSpec((8, 2048), lambda w, j: (w, j))],
      out_specs=pl.BlockSpec((8, 16, 128), lambda w, j: (w, j, 0)),
      out_shape=jax.ShapeDtypeStruct((grid_w * 8, S // 128, 128), jnp.int32),
  )


@functools.lru_cache(maxsize=None)
def _build_gather(B, V, D, chunk):
  NC, NS = 2, 16
  NW = NC * NS
  b_per_w = B // NW
  n_chunk = b_per_w // chunk
  raw_rows = b_per_w // 128
  ready_rows = b_per_w // chunk
  assert n_chunk % _NBUF == 0

  mesh = plsc.VectorSubcoreMesh(core_axis_name="c", subcore_axis_name="s")

  @functools.partial(
      pl.kernel,
      mesh=mesh,
      compiler_params=pltpu.CompilerParams(
          use_tc_tiling_on_sc=False, needs_layout_passes=False),
      out_type=jax.ShapeDtypeStruct((B, D), jnp.float32),
      scratch_types=[
          pltpu.VMEM((raw_rows, 128), jnp.int32),
          pltpu.VMEM((ready_rows, chunk), jnp.int32),
          pltpu.VMEM((_NBUF * chunk, D), jnp.float32),
          pltpu.SemaphoreType.DMA((_NBUF,)),
          pltpu.SemaphoreType.DMA((_NBUF,)),
      ],
  )
  def gather_kernel(idx_hbm, table_hbm, out_hbm, raw_v, idx_v, rows_v, gsem,
                    osem):
    wid = lax.axis_index("s") * NC + lax.axis_index("c")
    out_base_w = wid * b_per_w

    pltpu.sync_copy(idx_hbm.at[pl.ds(wid * raw_rows, raw_rows)], raw_v)

    i16 = lax.iota(jnp.int32, 16)
    pattern = ((i16 & 1) * (_WIN // 2)) + (i16 >> 1)
    vregs_per_row = chunk // 16

    def permute_row(r, carry):
      for v in range(vregs_per_row):
        k0 = r * chunk + v * 16
        w0 = k0 & ~(_WIN - 1)
        src = pattern + (w0 + ((k0 - w0) >> 1))
        val = plsc.load_gather(raw_v, [src >> 7, src & 127])
        idx_v[r, pl.ds(v * 16, 16)] = val
      return carry

    lax.fori_loop(0, ready_rows, permute_row, 0)

    def fire_gather(g, b):
      pltpu.async_copy(table_hbm.at[idx_v.at[g]],
                       rows_v.at[pl.ds(b * chunk, chunk)], gsem.at[b])

    def drain_gather(b):
      pltpu.make_async_copy(table_hbm.at[pl.ds(0, chunk)],
                            rows_v.at[pl.ds(b * chunk, chunk)],
                            gsem.at[b]).wait()

    def wait_outwrite(b):
      pltpu.make_async_copy(rows_v.at[pl.ds(b * chunk, chunk)],
                            out_hbm.at[pl.ds(out_base_w, chunk)],
                            osem.at[b]).wait()

    for b in range(_NBUF):
      fire_gather(b, b)

    def body(t, carry):
      for b in range(_NBUF):
        g = _NBUF * t + b
        drain_gather(b)
        pltpu.async_copy(rows_v.at[pl.ds(b * chunk, chunk)],
                         out_hbm.at[pl.ds(out_base_w + g * chunk, chunk)],
                         osem.at[b])
      for b in range(_NBUF):
        g_next = _NBUF * t + b + _NBUF

        @pl.when(g_next < n_chunk)
        def _():
          wait_outwrite(b)
          fire_gather(g_next, b)

      return carry

    lax.fori_loop(0, n_chunk // _NBUF, body, 0)
    for b in range(_NBUF):
      wait_outwrite(b)

  return gather_kernel


def _retile_body(x_ref, o_ref):
  x = x_ref[...][0]
  h = _WIN // 2
  for u in range(x.shape[0] // h):
    win = x[u * h:(u + 1) * h]
    o_ref[0, :, u * _WIN:u * _WIN + h] = win[:, 0:64].T
    o_ref[0, :, u * _WIN + h:(u + 1) * _WIN] = win[:, 64:128].T


@functools.lru_cache(maxsize=None)
def _build_retile(S, W, D, sblk):
  grid_s = S // sblk
  return pl.pallas_call(
      _retile_body,
      grid=(W, grid_s),
      in_specs=[
          pl.BlockSpec((1, sblk * D // 128, 128), lambda w, j: (w, j, 0))
      ],
      out_specs=pl.BlockSpec((1, D, sblk), lambda w, j: (w, 0, j)),
      out_shape=jax.ShapeDtypeStruct((W, D, S), jnp.float32),
  )


def kernel(input_sentence, word_embedding_weight):
  S, W = input_sentence.shape
  V, D = word_embedding_weight.shape
  B = S * W

  wt = word_embedding_weight.T
  t2d = _build_detile(V, D, _COLS)(wt)
  Vp = t2d.shape[0] * 2
  table_rm = t2d.reshape(Vp * D // 128 * 128).reshape(Vp, D)

  idxT = input_sentence.T.astype(jnp.int32)
  half_bits = (_COLS // 2).bit_length() - 1
  u = (jnp.bitwise_and(idxT, -_COLS)
       + ((idxT & (_COLS // 2 - 1)) << 1)
       + ((idxT >> half_bits) & 1))

  idx3 = _build_idx(S, W)(u)
  idx2d = idx3.reshape(idx3.shape[0] * S // 128 * 128).reshape(-1, 128)

  out_sc = _build_gather(B, Vp, D, _CHUNK)(idx2d, table_rm)

  out_v = out_sc.reshape(B * D).reshape(W, S * D // 128, 128)
  out_t = _build_retile(S, W, D, _SBLK)(out_v)
  return out_t.transpose(2, 0, 1)


# speedup vs baseline: 2.0155x; 1.0506x over previous
import functools

import jax
import jax.numpy as jnp
from jax import lax
from jax.experimental import pallas as pl
from jax.experimental.pallas import tpu as pltpu
from jax.experimental.pallas import tpu_sc as plsc

_COLS = 32768
_WIN = 1024
_SBLK = 16384
_CHUNK = 256
_NBUF = 4


def _detile_body(x_ref, o_ref):
  y = x_ref[...].T
  half = y.shape[0] // 2
  o_ref[...] = jnp.concatenate([y[0:half], y[half:]], axis=1)


@functools.lru_cache(maxsize=None)
def _build_detile(V, D, cols):
  grid = -(-V // cols)
  return pl.pallas_call(
      _detile_body,
      grid=(grid,),
      in_specs=[pl.BlockSpec((D, cols), lambda i: (0, i))],
      out_specs=pl.BlockSpec((cols // 2, 128), lambda i: (i, 0)),
      out_shape=jax.ShapeDtypeStruct((grid * cols // 2, 128), jnp.float32),
  )


def _idx_body(x_ref, o_ref):
  x = x_ref[...]
  o_ref[...] = jnp.concatenate(
      [x[:, 128 * t:128 * (t + 1)][:, None, :] for t in range(16)], axis=1)


@functools.lru_cache(maxsize=None)
def _build_idx(S, W):
  grid_w = -(-W // 8)
  grid_s = S // 2048
  return pl.pallas_call(
      _idx_body,
      grid=(grid_w, grid_s),
      in_specs=[pl.BlockSpec((8, 2048), lambda w, j: (w, j))],
      out_specs=pl.BlockSpec((8, 16, 128), lambda w, j: (w, j, 0)),
      out_shape=jax.ShapeDtypeStruct((grid_w * 8, S // 128, 128), jnp.int32),
  )


@functools.lru_cache(maxsize=None)
def _build_gather(B, V, D, chunk):
  NC, NS = 2, 16
  NW = NC * NS
  b_per_w = B // NW
  n_chunk = b_per_w // chunk
  raw_rows = b_per_w // 128
  ready_rows = b_per_w // chunk
  assert n_chunk % _NBUF == 0

  mesh = plsc.VectorSubcoreMesh(core_axis_name="c", subcore_axis_name="s")

  @functools.partial(
      pl.kernel,
      mesh=mesh,
      compiler_params=pltpu.CompilerParams(
          use_tc_tiling_on_sc=False, needs_layout_passes=False),
      out_type=jax.ShapeDtypeStruct((B, D), jnp.float32),
      scratch_types=[
          pltpu.VMEM((raw_rows, 128), jnp.int32),
          pltpu.VMEM((ready_rows, chunk), jnp.int32),
          pltpu.VMEM((_NBUF * chunk, D), jnp.float32),
          pltpu.SemaphoreType.DMA((_NBUF,)),
          pltpu.SemaphoreType.DMA((_NBUF,)),
      ],
  )
  def gather_kernel(idx_hbm, table_hbm, out_hbm, raw_v, idx_v, rows_v, gsem,
                    osem):
    wid = lax.axis_index("s") * NC + lax.axis_index("c")
    out_base_w = wid * b_per_w

    pltpu.sync_copy(idx_hbm.at[pl.ds(wid * raw_rows, raw_rows)], raw_v)

    i16 = lax.iota(jnp.int32, 16)
    pattern = ((i16 & 1) * (_WIN // 2)) + (i16 >> 1)
    vregs_per_row = chunk // 16

    def permute_row(r, carry):
      for v in range(vregs_per_row):
        k0 = r * chunk + v * 16
        w0 = k0 & ~(_WIN - 1)
        src = pattern + (w0 + ((k0 - w0) >> 1))
        val = plsc.load_gather(raw_v, [src >> 7, src & 127])
        idx_v[r, pl.ds(v * 16, 16)] = val
      return carry

    lax.fori_loop(0, ready_rows, permute_row, 0)

    def fire_gather(g, b):
      pltpu.async_copy(table_hbm.at[idx_v.at[g]],
                       rows_v.at[pl.ds(b * chunk, chunk)], gsem.at[b])

    def drain_gather(b):
      pltpu.make_async_copy(table_hbm.at[pl.ds(0, chunk)],
                            rows_v.at[pl.ds(b * chunk, chunk)],
                            gsem.at[b]).wait()

    def wait_outwrite(b):
      pltpu.make_async_copy(rows_v.at[pl.ds(b * chunk, chunk)],
                            out_hbm.at[pl.ds(out_base_w, chunk)],
                            osem.at[b]).wait()

    for b in range(_NBUF):
      fire_gather(b, b)

    def body(t, carry):
      for b in range(_NBUF):
        g = _NBUF * t + b
        drain_gather(b)
        pltpu.async_copy(rows_v.at[pl.ds(b * chunk, chunk)],
                         out_hbm.at[pl.ds(out_base_w + g * chunk, chunk)],
                         osem.at[b])
      for b in range(_NBUF):
        g_next = _NBUF * t + b + _NBUF

        @pl.when(g_next < n_chunk)
        def _():
          wait_outwrite(b)
          fire_gather(g_next, b)

      return carry

    lax.fori_loop(0, n_chunk // _NBUF, body, 0)
    for b in range(_NBUF):
      wait_outwrite(b)

  return gather_kernel


def _retile_body(x_ref, o_ref):
  xw = x_ref[...]
  h = _WIN // 2
  for w in range(xw.shape[0]):
    x = xw[w]
    for u in range(x.shape[0] // h):
      win = x[u * h:(u + 1) * h]
      o_ref[w, :, u * _WIN:u * _WIN + h] = win[:, 0:64].T
      o_ref[w, :, u * _WIN + h:(u + 1) * _WIN] = win[:, 64:128].T


@functools.lru_cache(maxsize=None)
def _build_retile(S, W, D, sblk, wblk):
  grid_s = S // sblk
  grid_w = -(-W // wblk)
  return pl.pallas_call(
      _retile_body,
      grid=(grid_w, grid_s),
      in_specs=[
          pl.BlockSpec((wblk, sblk * D // 128, 128), lambda w, j: (w, j, 0))
      ],
      out_specs=pl.BlockSpec((wblk, D, sblk), lambda w, j: (w, 0, j)),
      out_shape=jax.ShapeDtypeStruct((grid_w * wblk, D, S), jnp.float32),
  )


def kernel(input_sentence, word_embedding_weight):
  S, W = input_sentence.shape
  V, D = word_embedding_weight.shape
  B = S * W

  wt = word_embedding_weight.T
  t2d = _build_detile(V, D, _COLS)(wt)
  Vp = t2d.shape[0] * 2
  table_rm = t2d.reshape(Vp * D // 128 * 128).reshape(Vp, D)

  idxT = input_sentence.T.astype(jnp.int32)
  half_bits = (_COLS // 2).bit_length() - 1
  u = (jnp.bitwise_and(idxT, -_COLS)
       + ((idxT & (_COLS // 2 - 1)) << 1)
       + ((idxT >> half_bits) & 1))

  idx3 = _build_idx(S, W)(u)
  idx2d = idx3.reshape(idx3.shape[0] * S // 128 * 128).reshape(-1, 128)

  out_sc = _build_gather(B, Vp, D, _CHUNK)(idx2d, table_rm)

  out_v = out_sc.reshape(B * D).reshape(W, S * D // 128, 128)
  out_t = _build_retile(S, W, D, _SBLK, 2)(out_v)
  return out_t[0:W].transpose(2, 0, 1)


# speedup vs baseline: 2.0899x; 1.0369x over previous
import functools

import jax
import jax.numpy as jnp
from jax import lax
from jax.experimental import pallas as pl
from jax.experimental.pallas import tpu as pltpu
from jax.experimental.pallas import tpu_sc as plsc

_COLS = 32768
_WIN = 1024
_SBLK = 16384
_CHUNK = 256
_NBUF = 4


def _detile_body(x_ref, o_ref):
  y = x_ref[...].T
  half = y.shape[0] // 2
  o_ref[...] = jnp.concatenate([y[0:half], y[half:]], axis=1)


@functools.lru_cache(maxsize=None)
def _build_detile(V, D, cols):
  grid = -(-V // cols)
  return pl.pallas_call(
      _detile_body,
      grid=(grid,),
      in_specs=[pl.BlockSpec((D, cols), lambda i: (0, i))],
      out_specs=pl.BlockSpec((cols // 2, 128), lambda i: (i, 0)),
      out_shape=jax.ShapeDtypeStruct((grid * cols // 2, 128), jnp.float32),
  )


def _idx_body(x_ref, o_ref):
  x = x_ref[...]
  n = x.shape[1] // 128
  o_ref[...] = jnp.concatenate(
      [x[:, 128 * t:128 * (t + 1)][:, None, :] for t in range(n)], axis=1)


@functools.lru_cache(maxsize=None)
def _build_idx(S, W):
  grid_w = -(-W // 8)
  return pl.pallas_call(
      _idx_body,
      grid=(grid_w,),
      in_specs=[pl.BlockSpec((8, S), lambda w: (w, 0))],
      out_specs=pl.BlockSpec((8, S // 128, 128), lambda w: (w, 0, 0)),
      out_shape=jax.ShapeDtypeStruct((grid_w * 8, S // 128, 128), jnp.int32),
  )


@functools.lru_cache(maxsize=None)
def _build_gather(B, V, D, chunk):
  NC, NS = 2, 16
  NW = NC * NS
  b_per_w = B // NW
  n_chunk = b_per_w // chunk
  raw_rows = b_per_w // 128
  ready_rows = b_per_w // chunk
  assert n_chunk % _NBUF == 0

  mesh = plsc.VectorSubcoreMesh(core_axis_name="c", subcore_axis_name="s")

  @functools.partial(
      pl.kernel,
      mesh=mesh,
      compiler_params=pltpu.CompilerParams(
          use_tc_tiling_on_sc=False, needs_layout_passes=False),
      out_type=jax.ShapeDtypeStruct((B, D), jnp.float32),
      scratch_types=[
          pltpu.VMEM((raw_rows, 128), jnp.int32),
          pltpu.VMEM((ready_rows, chunk), jnp.int32),
          pltpu.VMEM((_NBUF * chunk, D), jnp.float32),
          pltpu.SemaphoreType.DMA((_NBUF,)),
          pltpu.SemaphoreType.DMA((_NBUF,)),
      ],
  )
  def gather_kernel(idx_hbm, table_hbm, out_hbm, raw_v, idx_v, rows_v, gsem,
                    osem):
    wid = lax.axis_index("s") * NC + lax.axis_index("c")
    out_base_w = wid * b_per_w

    pltpu.sync_copy(idx_hbm.at[pl.ds(wid * raw_rows, raw_rows)], raw_v)

    i16 = lax.iota(jnp.int32, 16)
    pattern = ((i16 & 1) * (_WIN // 2)) + (i16 >> 1)
    vregs_per_row = chunk // 16

    def permute_row(r, carry):
      for v in range(vregs_per_row):
        k0 = r * chunk + v * 16
        w0 = k0 & ~(_WIN - 1)
        src = pattern + (w0 + ((k0 - w0) >> 1))
        val = plsc.load_gather(raw_v, [src >> 7, src & 127])
        idx_v[r, pl.ds(v * 16, 16)] = val
      return carry

    lax.fori_loop(0, ready_rows, permute_row, 0)

    def fire_gather(g, b):
      pltpu.async_copy(table_hbm.at[idx_v.at[g]],
                       rows_v.at[pl.ds(b * chunk, chunk)], gsem.at[b])

    def drain_gather(b):
      pltpu.make_async_copy(table_hbm.at[pl.ds(0, chunk)],
                            rows_v.at[pl.ds(b * chunk, chunk)],
                            gsem.at[b]).wait()

    def wait_outwrite(b):
      pltpu.make_async_copy(rows_v.at[pl.ds(b * chunk, chunk)],
                            out_hbm.at[pl.ds(out_base_w, chunk)],
                            osem.at[b]).wait()

    for b in range(_NBUF):
      fire_gather(b, b)

    def body(t, carry):
      for b in range(_NBUF):
        g = _NBUF * t + b
        drain_gather(b)
        pltpu.async_copy(rows_v.at[pl.ds(b * chunk, chunk)],
                         out_hbm.at[pl.ds(out_base_w + g * chunk, chunk)],
                         osem.at[b])
      for b in range(_NBUF):
        g_next = _NBUF * t + b + _NBUF

        @pl.when(g_next < n_chunk)
        def _():
          wait_outwrite(b)
          fire_gather(g_next, b)

      return carry

    lax.fori_loop(0, n_chunk // _NBUF, body, 0)
    for b in range(_NBUF):
      wait_outwrite(b)

  return gather_kernel


def _retile_body(x_ref, o_ref):
  xw = x_ref[...]
  h = _WIN // 2
  for w in range(xw.shape[0]):
    x = xw[w]
    for u in range(x.shape[0] // h):
      win = x[u * h:(u + 1) * h]
      o_ref[w, :, u * _WIN:u * _WIN + h] = win[:, 0:64].T
      o_ref[w, :, u * _WIN + h:(u + 1) * _WIN] = win[:, 64:128].T


@functools.lru_cache(maxsize=None)
def _build_retile(S, W, D, sblk, wblk):
  grid_s = S // sblk
  grid_w = -(-W // wblk)
  return pl.pallas_call(
      _retile_body,
      grid=(grid_w, grid_s),
      in_specs=[
          pl.BlockSpec((wblk, sblk * D // 128, 128), lambda w, j: (w, j, 0))
      ],
      out_specs=pl.BlockSpec((wblk, D, sblk), lambda w, j: (w, 0, j)),
      out_shape=jax.ShapeDtypeStruct((grid_w * wblk, D, S), jnp.float32),
  )


def kernel(input_sentence, word_embedding_weight):
  S, W = input_sentence.shape
  V, D = word_embedding_weight.shape
  B = S * W

  wt = word_embedding_weight.T
  t2d = _build_detile(V, D, _COLS)(wt)
  Vp = t2d.shape[0] * 2
  table_rm = t2d.reshape(Vp * D // 128 * 128).reshape(Vp, D)

  idxT = input_sentence.T.astype(jnp.int32)
  half_bits = (_COLS // 2).bit_length() - 1
  u = (jnp.bitwise_and(idxT, -_COLS)
       + ((idxT & (_COLS // 2 - 1)) << 1)
       + ((idxT >> half_bits) & 1))

  idx3 = _build_idx(S, W)(u)
  idx2d = idx3.reshape(idx3.shape[0] * S // 128 * 128).reshape(-1, 128)

  out_sc = _build_gather(B, Vp, D, _CHUNK)(idx2d, table_rm)

  out_v = out_sc.reshape(B * D).reshape(W, S * D // 128, 128)
  out_t = _build_retile(S, W, D, _SBLK, 2)(out_v)
  return out_t[0:W].transpose(2, 0, 1)
